# baseline (speedup 1.0000x reference)
import functools
import math

import jax
import jax.numpy as jnp
from jax.experimental import pallas as pl
from jax.experimental.pallas import tpu as pltpu

FD = 256
HEADS = 8
HDIM = FD // HEADS
NBATCH = 16
TK = 128
_SCALE = 1.0 / math.sqrt(HDIM)
_NEG = -1e30


def _proj_body(h_ref, wk_ref, wv_ref, k_ref, v_ref):
    h = h_ref[...]
    k_ref[...] = jnp.dot(h, wk_ref[...], preferred_element_type=jnp.float32)
    v_ref[...] = jnp.dot(h, wv_ref[...], preferred_element_type=jnp.float32)


def _project_kv(h, wk, wv, tq=256):
    n = h.shape[0]
    bs_h = pl.BlockSpec((tq, FD), lambda i: (i, 0))
    bs_w = pl.BlockSpec((FD, FD), lambda i: (0, 0))
    return pl.pallas_call(
        _proj_body,
        grid=(n // tq,),
        in_specs=[bs_h, bs_w, bs_w],
        out_specs=[bs_h, bs_h],
        out_shape=[jax.ShapeDtypeStruct((n, FD), jnp.float32)] * 2,
    )(h, wk, wv)


def _attn_body(lohi_ref, h_ref, qb_ref, kb_ref, k_ref, v_ref,
               wq_ref, wgh_ref, wgc_ref, bg_ref, wu_ref, bu_ref,
               g_ref, b_ref, w1_ref, b1_ref, w2_ref, b2_ref, out_ref):
    i = pl.program_id(0)
    lo = lohi_ref[2 * i]
    hi = lohi_ref[2 * i + 1]
    h = h_ref[...]
    tq = h.shape[0]
    q = jnp.dot(h, wq_ref[...], preferred_element_type=jnp.float32) * _SCALE
    qb = qb_ref[...]

    ctx_parts = []
    for hd in range(HEADS):
        qh = q[:, hd * HDIM:(hd + 1) * HDIM]

        def body(j, carry, qh=qh, hd=hd):
            m, l, acc = carry
            kh = k_ref[pl.ds(j * TK, TK), hd * HDIM:(hd + 1) * HDIM]
            vh = v_ref[pl.ds(j * TK, TK), hd * HDIM:(hd + 1) * HDIM]
            kb = kb_ref[pl.ds(j, 1), :]
            s = jax.lax.dot_general(qh, kh, (((1,), (1,)), ((), ())),
                                    preferred_element_type=jnp.float32)
            mask = qb == kb
            s = jnp.where(mask, s, _NEG)
            m_new = jnp.maximum(m, s.max(axis=1, keepdims=True))
            p = jnp.where(mask, jnp.exp(s - m_new), 0.0)
            alpha = jnp.exp(m - m_new)
            l_new = l * alpha + p.sum(axis=1, keepdims=True)
            acc_new = acc * alpha + jnp.dot(p, vh,
                                            preferred_element_type=jnp.float32)
            return m_new, l_new, acc_new

        m0 = jnp.full((tq, 1), _NEG, jnp.float32)
        l0 = jnp.zeros((tq, 1), jnp.float32)
        a0 = jnp.zeros((tq, HDIM), jnp.float32)
        m, l, acc = jax.lax.fori_loop(lo, hi, body, (m0, l0, a0))
        ctx_parts.append(acc / jnp.where(l > 0.0, l, 1.0))
    ctx = jnp.concatenate(ctx_parts, axis=1)

    gate = jax.nn.sigmoid(
        jnp.dot(h, wgh_ref[...], preferred_element_type=jnp.float32)
        + jnp.dot(ctx, wgc_ref[...], preferred_element_type=jnp.float32)
        + bg_ref[...])
    hu = h + gate * (jnp.dot(ctx, wu_ref[...],
                             preferred_element_type=jnp.float32) + bu_ref[...])
    mean = jnp.mean(hu, axis=1, keepdims=True)
    var = jnp.mean((hu - mean) ** 2, axis=1, keepdims=True)
    y = (hu - mean) / jnp.sqrt(var + 1e-5) * g_ref[...] + b_ref[...]
    z = jnp.maximum(
        jnp.dot(y, w1_ref[...], preferred_element_type=jnp.float32)
        + b1_ref[...], 0.0)
    out_ref[...] = hu + jnp.dot(z, w2_ref[...],
                                preferred_element_type=jnp.float32) + b2_ref[...]


def _attn_update(h, q_batch, k_batch, kmat, vmat, wq, wg, bg, wu, bu,
                 ln_g, ln_b, w1, b1, w2, b2, lohi, tq):
    nq = h.shape[0]
    nk = kmat.shape[0]
    nkt = nk // TK
    wgh = wg[:FD]
    wgc = wg[FD:]
    full = lambda shape: pl.BlockSpec(shape, lambda i, s: (0, 0))
    grid_spec = pltpu.PrefetchScalarGridSpec(
        num_scalar_prefetch=1,
        grid=(nq // tq,),
        in_specs=[
            pl.BlockSpec((tq, FD), lambda i, s: (i, 0)),
            pl.BlockSpec((tq, 1), lambda i, s: (i, 0)),
            full((nkt, TK)),
            full((nk, FD)),
            full((nk, FD)),
            full((FD, FD)),
            full((FD, FD)),
            full((FD, FD)),
            full((1, FD)),
            full((FD, FD)),
            full((1, FD)),
            full((1, FD)),
            full((1, FD)),
            full((FD, 4 * FD)),
            full((1, 4 * FD)),
            full((4 * FD, FD)),
            full((1, FD)),
        ],
        out_specs=pl.BlockSpec((tq, FD), lambda i, s: (i, 0)),
    )
    return pl.pallas_call(
        _attn_body,
        grid_spec=grid_spec,
        out_shape=jax.ShapeDtypeStruct((nq, FD), jnp.float32),
    )(lohi, h, q_batch.reshape(nq, 1), k_batch.reshape(nkt, TK), kmat, vmat,
      wq, wgh, wgc, bg.reshape(1, FD), wu, bu.reshape(1, FD),
      ln_g.reshape(1, FD), ln_b.reshape(1, FD),
      w1, b1.reshape(1, 4 * FD), w2, b2.reshape(1, FD))


def _tile_bounds(q_batch, k_starts, k_ends, tq):
    qb = q_batch.reshape(-1, tq)
    bmin = qb[:, 0]
    bmax = qb[:, -1]
    lo = (k_starts[bmin] // TK).astype(jnp.int32)
    hi = ((k_ends[bmax] + TK - 1) // TK).astype(jnp.int32)
    return jnp.stack([lo, hi], axis=1).reshape(-1)


def kernel(h_protein, h_ligand, protein_batch, ligand_batch, wq_l, wk_p, wv_p,
           wg_l, bg_l, wu_l, bu_l, wq_p, wk_l, wv_l, wg_p, bg_p, wu_p, bu_p,
           ln_p_g, ln_p_b, ln_l_g, ln_l_b, fp_w1, fp_b1, fp_w2, fp_b2,
           fl_w1, fl_b1, fl_w2, fl_b2):
    bvec = jnp.arange(NBATCH, dtype=protein_batch.dtype)
    p_start = jnp.sum(protein_batch[None, :] < bvec[:, None], axis=1)
    p_end = jnp.sum(protein_batch[None, :] <= bvec[:, None], axis=1)
    l_start = jnp.sum(ligand_batch[None, :] < bvec[:, None], axis=1)
    l_end = jnp.sum(ligand_batch[None, :] <= bvec[:, None], axis=1)

    tq_l = 128
    tq_p = 128
    lohi_l = _tile_bounds(ligand_batch, p_start, p_end, tq_l)
    lohi_p = _tile_bounds(protein_batch, l_start, l_end, tq_p)

    k_p, v_p = _project_kv(h_protein, wk_p, wv_p)
    k_l, v_l = _project_kv(h_ligand, wk_l, wv_l)

    l_final = _attn_update(h_ligand, ligand_batch, protein_batch, k_p, v_p,
                           wq_l, wg_l, bg_l, wu_l, bu_l, ln_l_g, ln_l_b,
                           fl_w1, fl_b1, fl_w2, fl_b2, lohi_l, tq_l)
    p_final = _attn_update(h_protein, protein_batch, ligand_batch, k_l, v_l,
                           wq_p, wg_p, bg_p, wu_p, bu_p, ln_p_g, ln_p_b,
                           fp_w1, fp_b1, fp_w2, fp_b2, lohi_p, tq_p)
    return (p_final, l_final)


# speedup vs baseline: 1.3669x; 1.3669x over previous
import math

import jax
import jax.numpy as jnp
from jax.experimental import pallas as pl
from jax.experimental.pallas import tpu as pltpu

FD = 256
HEADS = 8
HDIM = FD // HEADS
NBATCH = 16
TK = 128
_SCALE = 1.0 / math.sqrt(HDIM)
_NEG = -1e30


def _proj_body(h_ref, wk_ref, wv_ref, k_ref, v_ref):
    h = h_ref[...].astype(jnp.bfloat16)
    k_ref[...] = jnp.dot(h, wk_ref[...],
                         preferred_element_type=jnp.float32).astype(jnp.bfloat16)
    v_ref[...] = jnp.dot(h, wv_ref[...],
                         preferred_element_type=jnp.float32).astype(jnp.bfloat16)


def _project_kv(h, wk, wv, tq=256):
    n = h.shape[0]
    bs_h = pl.BlockSpec((tq, FD), lambda i: (i, 0))
    bs_w = pl.BlockSpec((FD, FD), lambda i: (0, 0))
    return pl.pallas_call(
        _proj_body,
        grid=(n // tq,),
        in_specs=[bs_h, bs_w, bs_w],
        out_specs=[bs_h, bs_h],
        out_shape=[jax.ShapeDtypeStruct((n, FD), jnp.bfloat16)] * 2,
    )(h, wk.astype(jnp.bfloat16), wv.astype(jnp.bfloat16))


def _attn_body(lohi_ref, h_ref, qb_ref, kb_ref, k_ref, v_ref,
               wq_ref, wgh_ref, wgc_ref, bg_ref, wu_ref, bu_ref,
               g_ref, b_ref, w1_ref, b1_ref, w2_ref, b2_ref, out_ref):
    i = pl.program_id(0)
    lo = lohi_ref[2 * i]
    hi = lohi_ref[2 * i + 1]
    h = h_ref[...]
    tq = h.shape[0]
    hb = h.astype(jnp.bfloat16)
    q = jnp.dot(hb, wq_ref[...], preferred_element_type=jnp.float32) * _SCALE
    qbf = q.astype(jnp.bfloat16)
    qhs = [qbf[:, hd * HDIM:(hd + 1) * HDIM] for hd in range(HEADS)]
    qb = qb_ref[...]

    def body(j, carry):
        ms, ls, accs = carry
        kb = kb_ref[pl.ds(j, 1), :]
        mask = qb == kb
        ktile = k_ref[pl.ds(j * TK, TK), :]
        vtile = v_ref[pl.ds(j * TK, TK), :]
        nms, nls, naccs = [], [], []
        for hd in range(HEADS):
            kh = ktile[:, hd * HDIM:(hd + 1) * HDIM]
            vh = vtile[:, hd * HDIM:(hd + 1) * HDIM]
            s = jax.lax.dot_general(qhs[hd], kh, (((1,), (1,)), ((), ())),
                                    preferred_element_type=jnp.float32)
            s = jnp.where(mask, s, _NEG)
            m_new = jnp.maximum(ms[hd], s.max(axis=1, keepdims=True))
            p = jnp.where(mask, jnp.exp(s - m_new), 0.0)
            alpha = jnp.exp(ms[hd] - m_new)
            nls.append(ls[hd] * alpha + p.sum(axis=1, keepdims=True))
            pv = jax.lax.dot_general(p.astype(jnp.bfloat16), vh,
                                     (((1,), (0,)), ((), ())),
                                     preferred_element_type=jnp.float32)
            naccs.append(accs[hd] * alpha + pv)
            nms.append(m_new)
        return nms, nls, naccs

    m0 = [jnp.full((tq, 1), _NEG, jnp.float32)] * HEADS
    l0 = [jnp.zeros((tq, 1), jnp.float32)] * HEADS
    a0 = [jnp.zeros((tq, HDIM), jnp.float32)] * HEADS
    ms, ls, accs = jax.lax.fori_loop(lo, hi, body, (m0, l0, a0))
    ctx = jnp.concatenate(
        [accs[hd] / jnp.where(ls[hd] > 0.0, ls[hd], 1.0) for hd in range(HEADS)],
        axis=1)
    ctxb = ctx.astype(jnp.bfloat16)

    gate = jax.nn.sigmoid(
        jnp.dot(hb, wgh_ref[...], preferred_element_type=jnp.float32)
        + jnp.dot(ctxb, wgc_ref[...], preferred_element_type=jnp.float32)
        + bg_ref[...])
    hu = h + gate * (jnp.dot(ctxb, wu_ref[...],
                             preferred_element_type=jnp.float32) + bu_ref[...])
    mean = jnp.mean(hu, axis=1, keepdims=True)
    var = jnp.mean((hu - mean) ** 2, axis=1, keepdims=True)
    y = (hu - mean) / jnp.sqrt(var + 1e-5) * g_ref[...] + b_ref[...]
    z = jnp.maximum(
        jnp.dot(y.astype(jnp.bfloat16), w1_ref[...],
                preferred_element_type=jnp.float32) + b1_ref[...], 0.0)
    out_ref[...] = hu + jnp.dot(z.astype(jnp.bfloat16), w2_ref[...],
                                preferred_element_type=jnp.float32) + b2_ref[...]


def _attn_update(h, q_batch, k_batch, kmat, vmat, wq, wg, bg, wu, bu,
                 ln_g, ln_b, w1, b1, w2, b2, lohi, tq):
    nq = h.shape[0]
    nk = kmat.shape[0]
    nkt = nk // TK
    bf = jnp.bfloat16
    wgh = wg[:FD].astype(bf)
    wgc = wg[FD:].astype(bf)
    full = lambda shape: pl.BlockSpec(shape, lambda i, s: (0, 0))
    grid_spec = pltpu.PrefetchScalarGridSpec(
        num_scalar_prefetch=1,
        grid=(nq // tq,),
        in_specs=[
            pl.BlockSpec((tq, FD), lambda i, s: (i, 0)),
            pl.BlockSpec((tq, 1), lambda i, s: (i, 0)),
            full((nkt, TK)),
            full((nk, FD)),
            full((nk, FD)),
            full((FD, FD)),
            full((FD, FD)),
            full((FD, FD)),
            full((1, FD)),
            full((FD, FD)),
            full((1, FD)),
            full((1, FD)),
            full((1, FD)),
            full((FD, 4 * FD)),
            full((1, 4 * FD)),
            full((4 * FD, FD)),
            full((1, FD)),
        ],
        out_specs=pl.BlockSpec((tq, FD), lambda i, s: (i, 0)),
    )
    return pl.pallas_call(
        _attn_body,
        grid_spec=grid_spec,
        out_shape=jax.ShapeDtypeStruct((nq, FD), jnp.float32),
    )(lohi, h, q_batch.reshape(nq, 1), k_batch.reshape(nkt, TK), kmat, vmat,
      wq.astype(bf), wgh, wgc, bg.reshape(1, FD), wu.astype(bf),
      bu.reshape(1, FD), ln_g.reshape(1, FD), ln_b.reshape(1, FD),
      w1.astype(bf), b1.reshape(1, 4 * FD), w2.astype(bf), b2.reshape(1, FD))


def _tile_bounds(q_batch, k_starts, k_ends, tq):
    qb = q_batch.reshape(-1, tq)
    bmin = qb[:, 0]
    bmax = qb[:, -1]
    lo = (k_starts[bmin] // TK).astype(jnp.int32)
    hi = ((k_ends[bmax] + TK - 1) // TK).astype(jnp.int32)
    return jnp.stack([lo, hi], axis=1).reshape(-1)


def kernel(h_protein, h_ligand, protein_batch, ligand_batch, wq_l, wk_p, wv_p,
           wg_l, bg_l, wu_l, bu_l, wq_p, wk_l, wv_l, wg_p, bg_p, wu_p, bu_p,
           ln_p_g, ln_p_b, ln_l_g, ln_l_b, fp_w1, fp_b1, fp_w2, fp_b2,
           fl_w1, fl_b1, fl_w2, fl_b2):
    bvec = jnp.arange(NBATCH, dtype=protein_batch.dtype)
    p_start = jnp.sum(protein_batch[None, :] < bvec[:, None], axis=1)
    p_end = jnp.sum(protein_batch[None, :] <= bvec[:, None], axis=1)
    l_start = jnp.sum(ligand_batch[None, :] < bvec[:, None], axis=1)
    l_end = jnp.sum(ligand_batch[None, :] <= bvec[:, None], axis=1)

    tq_l = 128
    tq_p = 128
    lohi_l = _tile_bounds(ligand_batch, p_start, p_end, tq_l)
    lohi_p = _tile_bounds(protein_batch, l_start, l_end, tq_p)

    k_p, v_p = _project_kv(h_protein, wk_p, wv_p)
    k_l, v_l = _project_kv(h_ligand, wk_l, wv_l)

    l_final = _attn_update(h_ligand, ligand_batch, protein_batch, k_p, v_p,
                           wq_l, wg_l, bg_l, wu_l, bu_l, ln_l_g, ln_l_b,
                           fl_w1, fl_b1, fl_w2, fl_b2, lohi_l, tq_l)
    p_final = _attn_update(h_protein, protein_batch, ligand_batch, k_l, v_l,
                           wq_p, wg_p, bg_p, wu_p, bu_p, ln_p_g, ln_p_b,
                           fp_w1, fp_b1, fp_w2, fp_b2, lohi_p, tq_p)
    return (p_final, l_final)


# speedup vs baseline: 1.3792x; 1.0090x over previous
import math

import jax
import jax.numpy as jnp
from jax.experimental import pallas as pl
from jax.experimental.pallas import tpu as pltpu

FD = 256
HEADS = 8
HDIM = FD // HEADS
NBATCH = 16
TK = 128
_SCALE = 1.0 / math.sqrt(HDIM)
_NEG = -1e30


def _proj_body(h_ref, wk_ref, wv_ref, k_ref, v_ref):
    h = h_ref[...].astype(jnp.bfloat16)
    k_ref[...] = jnp.dot(h, wk_ref[...],
                         preferred_element_type=jnp.float32).astype(jnp.bfloat16)
    v_ref[...] = jnp.dot(h, wv_ref[...],
                         preferred_element_type=jnp.float32).astype(jnp.bfloat16)


def _project_kv(h, wk, wv, tq=256):
    n = h.shape[0]
    bs_h = pl.BlockSpec((tq, FD), lambda i: (i, 0))
    bs_w = pl.BlockSpec((FD, FD), lambda i: (0, 0))
    return pl.pallas_call(
        _proj_body,
        grid=(n // tq,),
        in_specs=[bs_h, bs_w, bs_w],
        out_specs=[bs_h, bs_h],
        out_shape=[jax.ShapeDtypeStruct((n, FD), jnp.bfloat16)] * 2,
    )(h, wk.astype(jnp.bfloat16), wv.astype(jnp.bfloat16))


def _attn_body(lohi_ref, h_ref, qb_ref, kb_ref, k_ref, v_ref,
               wq_ref, wgh_ref, wgc_ref, bg_ref, wu_ref, bu_ref,
               g_ref, b_ref, w1_ref, b1_ref, w2_ref, b2_ref, out_ref,
               acc_ref):
    i = pl.program_id(0)
    lo = lohi_ref[2 * i]
    hi = lohi_ref[2 * i + 1]
    h = h_ref[...]
    tq = h.shape[0]
    hb = h.astype(jnp.bfloat16)
    q = jnp.dot(hb, wq_ref[...], preferred_element_type=jnp.float32) * _SCALE
    qbf = q.astype(jnp.bfloat16)
    qhs = [qbf[:, hd * HDIM:(hd + 1) * HDIM] for hd in range(HEADS)]
    qb = qb_ref[...]
    acc_ref[...] = jnp.zeros((tq, FD), jnp.float32)

    def body(j, carry):
        ms, ls = carry
        kb = kb_ref[pl.ds(j, 1), :]
        mask = qb == kb
        rows = pl.ds(j * TK, TK)
        nms, nls = [], []
        for hd in range(HEADS):
            sl = slice(hd * HDIM, (hd + 1) * HDIM)
            kh = k_ref[rows, sl]
            vh = v_ref[rows, sl]
            s = jax.lax.dot_general(qhs[hd], kh, (((1,), (1,)), ((), ())),
                                    preferred_element_type=jnp.float32)
            s = jnp.where(mask, s, _NEG)
            m_new = jnp.maximum(ms[hd], s.max(axis=1, keepdims=True))
            p = jnp.where(mask, jnp.exp(s - m_new), 0.0)
            alpha = jnp.exp(ms[hd] - m_new)
            nls.append(ls[hd] * alpha + p.sum(axis=1, keepdims=True))
            pv = jax.lax.dot_general(p.astype(jnp.bfloat16), vh,
                                     (((1,), (0,)), ((), ())),
                                     preferred_element_type=jnp.float32)
            acc_ref[:, sl] = acc_ref[:, sl] * alpha + pv
            nms.append(m_new)
        return nms, nls

    m0 = [jnp.full((tq, 1), _NEG, jnp.float32)] * HEADS
    l0 = [jnp.zeros((tq, 1), jnp.float32)] * HEADS
    ms, ls = jax.lax.fori_loop(lo, hi, body, (m0, l0))
    ctx = jnp.concatenate(
        [acc_ref[:, hd * HDIM:(hd + 1) * HDIM]
         / jnp.where(ls[hd] > 0.0, ls[hd], 1.0) for hd in range(HEADS)],
        axis=1)
    ctxb = ctx.astype(jnp.bfloat16)

    gate = jax.nn.sigmoid(
        jnp.dot(hb, wgh_ref[...], preferred_element_type=jnp.float32)
        + jnp.dot(ctxb, wgc_ref[...], preferred_element_type=jnp.float32)
        + bg_ref[...])
    hu = h + gate * (jnp.dot(ctxb, wu_ref[...],
                             preferred_element_type=jnp.float32) + bu_ref[...])
    mean = jnp.mean(hu, axis=1, keepdims=True)
    var = jnp.mean((hu - mean) ** 2, axis=1, keepdims=True)
    y = ((hu - mean) / jnp.sqrt(var + 1e-5) * g_ref[...]
         + b_ref[...]).astype(jnp.bfloat16)
    out = hu
    for c in range(4):
        cs = slice(c * FD, (c + 1) * FD)
        z = jnp.maximum(
            jnp.dot(y, w1_ref[:, cs], preferred_element_type=jnp.float32)
            + b1_ref[:, cs], 0.0)
        out = out + jnp.dot(z.astype(jnp.bfloat16), w2_ref[cs, :],
                            preferred_element_type=jnp.float32)
    out_ref[...] = out + b2_ref[...]


def _attn_update(h, q_batch, k_batch, kmat, vmat, wq, wg, bg, wu, bu,
                 ln_g, ln_b, w1, b1, w2, b2, lohi, tq):
    nq = h.shape[0]
    nk = kmat.shape[0]
    nkt = nk // TK
    bf = jnp.bfloat16
    wgh = wg[:FD].astype(bf)
    wgc = wg[FD:].astype(bf)
    full = lambda shape: pl.BlockSpec(shape, lambda i, s: (0, 0))
    grid_spec = pltpu.PrefetchScalarGridSpec(
        num_scalar_prefetch=1,
        grid=(nq // tq,),
        in_specs=[
            pl.BlockSpec((tq, FD), lambda i, s: (i, 0)),
            pl.BlockSpec((tq, 1), lambda i, s: (i, 0)),
            full((nkt, TK)),
            full((nk, FD)),
            full((nk, FD)),
            full((FD, FD)),
            full((FD, FD)),
            full((FD, FD)),
            full((1, FD)),
            full((FD, FD)),
            full((1, FD)),
            full((1, FD)),
            full((1, FD)),
            full((FD, 4 * FD)),
            full((1, 4 * FD)),
            full((4 * FD, FD)),
            full((1, FD)),
        ],
        out_specs=pl.BlockSpec((tq, FD), lambda i, s: (i, 0)),
        scratch_shapes=[pltpu.VMEM((tq, FD), jnp.float32)],
    )
    return pl.pallas_call(
        _attn_body,
        grid_spec=grid_spec,
        out_shape=jax.ShapeDtypeStruct((nq, FD), jnp.float32),
    )(lohi, h, q_batch.reshape(nq, 1), k_batch.reshape(nkt, TK), kmat, vmat,
      wq.astype(bf), wgh, wgc, bg.reshape(1, FD), wu.astype(bf),
      bu.reshape(1, FD), ln_g.reshape(1, FD), ln_b.reshape(1, FD),
      w1.astype(bf), b1.reshape(1, 4 * FD), w2.astype(bf), b2.reshape(1, FD))


def _tile_bounds(q_batch, k_starts, k_ends, tq):
    qb = q_batch.reshape(-1, tq)
    bmin = qb[:, 0]
    bmax = qb[:, -1]
    lo = (k_starts[bmin] // TK).astype(jnp.int32)
    hi = ((k_ends[bmax] + TK - 1) // TK).astype(jnp.int32)
    return jnp.stack([lo, hi], axis=1).reshape(-1)


def kernel(h_protein, h_ligand, protein_batch, ligand_batch, wq_l, wk_p, wv_p,
           wg_l, bg_l, wu_l, bu_l, wq_p, wk_l, wv_l, wg_p, bg_p, wu_p, bu_p,
           ln_p_g, ln_p_b, ln_l_g, ln_l_b, fp_w1, fp_b1, fp_w2, fp_b2,
           fl_w1, fl_b1, fl_w2, fl_b2):
    bvec = jnp.arange(NBATCH, dtype=protein_batch.dtype)
    p_start = jnp.sum(protein_batch[None, :] < bvec[:, None], axis=1)
    p_end = jnp.sum(protein_batch[None, :] <= bvec[:, None], axis=1)
    l_start = jnp.sum(ligand_batch[None, :] < bvec[:, None], axis=1)
    l_end = jnp.sum(ligand_batch[None, :] <= bvec[:, None], axis=1)

    tq_l = 128
    tq_p = 128
    lohi_l = _tile_bounds(ligand_batch, p_start, p_end, tq_l)
    lohi_p = _tile_bounds(protein_batch, l_start, l_end, tq_p)

    k_p, v_p = _project_kv(h_protein, wk_p, wv_p)
    k_l, v_l = _project_kv(h_ligand, wk_l, wv_l)

    l_final = _attn_update(h_ligand, ligand_batch, protein_batch, k_p, v_p,
                           wq_l, wg_l, bg_l, wu_l, bu_l, ln_l_g, ln_l_b,
                           fl_w1, fl_b1, fl_w2, fl_b2, lohi_l, tq_l)
    p_final = _attn_update(h_protein, protein_batch, ligand_batch, k_l, v_l,
                           wq_p, wg_p, bg_p, wu_p, bu_p, ln_p_g, ln_p_b,
                           fp_w1, fp_b1, fp_w2, fp_b2, lohi_p, tq_p)
    return (p_final, l_final)


# speedup vs baseline: 2.4031x; 1.7424x over previous
import math

import jax
import jax.numpy as jnp
from jax.experimental import pallas as pl
from jax.experimental.pallas import tpu as pltpu

FD = 256
HEADS = 8
HDIM = FD // HEADS
NBATCH = 16
_SCALE = 1.0 / math.sqrt(HDIM)
_NEG = -1e30


def _proj_body(h_ref, wk_ref, wv_ref, k_ref, v_ref):
    h = h_ref[...].astype(jnp.bfloat16)
    k_ref[...] = jnp.dot(h, wk_ref[...],
                         preferred_element_type=jnp.float32).astype(jnp.bfloat16)
    v_ref[...] = jnp.dot(h, wv_ref[...],
                         preferred_element_type=jnp.float32).astype(jnp.bfloat16)


def _project_kv(h, wk, wv, tq=256):
    n = h.shape[0]
    bs_h = pl.BlockSpec((tq, FD), lambda i: (i, 0))
    bs_w = pl.BlockSpec((FD, FD), lambda i: (0, 0))
    return pl.pallas_call(
        _proj_body,
        grid=(n // tq,),
        in_specs=[bs_h, bs_w, bs_w],
        out_specs=[bs_h, bs_h],
        out_shape=[jax.ShapeDtypeStruct((n, FD), jnp.bfloat16)] * 2,
    )(h, wk.astype(jnp.bfloat16), wv.astype(jnp.bfloat16))


def _make_attn_body(w):
    def _attn_body(lohi_ref, h_ref, qb_ref, kb_ref, k_ref, v_ref,
                   wq_ref, wgh_ref, wgc_ref, bg_ref, wu_ref, bu_ref,
                   g_ref, b_ref, w1_ref, b1_ref, w2_ref, b2_ref, out_ref,
                   acc_ref):
        i = pl.program_id(0)
        lo = lohi_ref[2 * i]
        hi = lohi_ref[2 * i + 1]
        h = h_ref[...]
        tq = h.shape[0]
        hb = h.astype(jnp.bfloat16)
        q = jnp.dot(hb, wq_ref[...],
                    preferred_element_type=jnp.float32) * _SCALE
        qbf = q.astype(jnp.bfloat16)
        qhs = [qbf[:, hd * HDIM:(hd + 1) * HDIM] for hd in range(HEADS)]
        qb = qb_ref[...]
        acc_ref[...] = jnp.zeros((tq, FD), jnp.float32)

        def body(j, carry):
            ms, ls = carry
            kb = kb_ref[pl.ds(j, 1), :]
            mask = qb == kb
            rows = pl.ds(j * w, w)
            nms, nls = [], []
            for hd in range(HEADS):
                sl = slice(hd * HDIM, (hd + 1) * HDIM)
                kh = k_ref[rows, sl]
                vh = v_ref[rows, sl]
                s = jax.lax.dot_general(qhs[hd], kh, (((1,), (1,)), ((), ())),
                                        preferred_element_type=jnp.float32)
                s = jnp.where(mask, s, _NEG)
                m_new = jnp.maximum(ms[hd], s.max(axis=1, keepdims=True))
                p = jnp.where(mask, jnp.exp(s - m_new), 0.0)
                alpha = jnp.exp(ms[hd] - m_new)
                nls.append(ls[hd] * alpha + p.sum(axis=1, keepdims=True))
                pv = jax.lax.dot_general(p.astype(jnp.bfloat16), vh,
                                         (((1,), (0,)), ((), ())),
                                         preferred_element_type=jnp.float32)
                acc_ref[:, sl] = acc_ref[:, sl] * alpha + pv
                nms.append(m_new)
            return nms, nls

        m0 = [jnp.full((tq, 1), _NEG, jnp.float32)] * HEADS
        l0 = [jnp.zeros((tq, 1), jnp.float32)] * HEADS
        ms, ls = jax.lax.fori_loop(lo, hi, body, (m0, l0))
        ctx = jnp.concatenate(
            [acc_ref[:, hd * HDIM:(hd + 1) * HDIM]
             / jnp.where(ls[hd] > 0.0, ls[hd], 1.0) for hd in range(HEADS)],
            axis=1)
        ctxb = ctx.astype(jnp.bfloat16)

        gate = jax.nn.sigmoid(
            jnp.dot(hb, wgh_ref[...], preferred_element_type=jnp.float32)
            + jnp.dot(ctxb, wgc_ref[...], preferred_element_type=jnp.float32)
            + bg_ref[...])
        hu = h + gate * (jnp.dot(ctxb, wu_ref[...],
                                 preferred_element_type=jnp.float32)
                         + bu_ref[...])
        mean = jnp.mean(hu, axis=1, keepdims=True)
        var = jnp.mean((hu - mean) ** 2, axis=1, keepdims=True)
        y = ((hu - mean) / jnp.sqrt(var + 1e-5) * g_ref[...]
             + b_ref[...]).astype(jnp.bfloat16)
        out = hu
        for c in range(4):
            cs = slice(c * FD, (c + 1) * FD)
            z = jnp.maximum(
                jnp.dot(y, w1_ref[:, cs], preferred_element_type=jnp.float32)
                + b1_ref[:, cs], 0.0)
            out = out + jnp.dot(z.astype(jnp.bfloat16), w2_ref[cs, :],
                                preferred_element_type=jnp.float32)
        out_ref[...] = out + b2_ref[...]

    return _attn_body


def _attn_update(h, q_batch, k_batch, kmat, vmat, wq, wg, bg, wu, bu,
                 ln_g, ln_b, w1, b1, w2, b2, lohi, tq, w):
    nq = h.shape[0]
    nk = kmat.shape[0]
    nkt = nk // w
    bf = jnp.bfloat16
    wgh = wg[:FD].astype(bf)
    wgc = wg[FD:].astype(bf)
    full = lambda shape: pl.BlockSpec(shape, lambda i, s: (0, 0))
    grid_spec = pltpu.PrefetchScalarGridSpec(
        num_scalar_prefetch=1,
        grid=(nq // tq,),
        in_specs=[
            pl.BlockSpec((tq, FD), lambda i, s: (i, 0)),
            pl.BlockSpec((tq, 1), lambda i, s: (i, 0)),
            full((nkt, w)),
            full((nk, FD)),
            full((nk, FD)),
            full((FD, FD)),
            full((FD, FD)),
            full((FD, FD)),
            full((1, FD)),
            full((FD, FD)),
            full((1, FD)),
            full((1, FD)),
            full((1, FD)),
            full((FD, 4 * FD)),
            full((1, 4 * FD)),
            full((4 * FD, FD)),
            full((1, FD)),
        ],
        out_specs=pl.BlockSpec((tq, FD), lambda i, s: (i, 0)),
        scratch_shapes=[pltpu.VMEM((tq, FD), jnp.float32)],
    )
    return pl.pallas_call(
        _make_attn_body(w),
        grid_spec=grid_spec,
        out_shape=jax.ShapeDtypeStruct((nq, FD), jnp.float32),
    )(lohi, h, q_batch.reshape(nq, 1), k_batch.reshape(nkt, w), kmat, vmat,
      wq.astype(bf), wgh, wgc, bg.reshape(1, FD), wu.astype(bf),
      bu.reshape(1, FD), ln_g.reshape(1, FD), ln_b.reshape(1, FD),
      w1.astype(bf), b1.reshape(1, 4 * FD), w2.astype(bf), b2.reshape(1, FD))


def _tile_bounds(q_batch, k_starts, k_ends, tq, w):
    qb = q_batch.reshape(-1, tq)
    bmin = qb[:, 0]
    bmax = qb[:, -1]
    lo = (k_starts[bmin] // w).astype(jnp.int32)
    hi = ((k_ends[bmax] + w - 1) // w).astype(jnp.int32)
    return jnp.stack([lo, hi], axis=1).reshape(-1)


def kernel(h_protein, h_ligand, protein_batch, ligand_batch, wq_l, wk_p, wv_p,
           wg_l, bg_l, wu_l, bu_l, wq_p, wk_l, wv_l, wg_p, bg_p, wu_p, bu_p,
           ln_p_g, ln_p_b, ln_l_g, ln_l_b, fp_w1, fp_b1, fp_w2, fp_b2,
           fl_w1, fl_b1, fl_w2, fl_b2):
    bvec = jnp.arange(NBATCH, dtype=protein_batch.dtype)
    p_start = jnp.sum(protein_batch[None, :] < bvec[:, None], axis=1)
    p_end = jnp.sum(protein_batch[None, :] <= bvec[:, None], axis=1)
    l_start = jnp.sum(ligand_batch[None, :] < bvec[:, None], axis=1)
    l_end = jnp.sum(ligand_batch[None, :] <= bvec[:, None], axis=1)

    tq_l, w_l = 256, 1024
    tq_p, w_p = 256, 256
    lohi_l = _tile_bounds(ligand_batch, p_start, p_end, tq_l, w_l)
    lohi_p = _tile_bounds(protein_batch, l_start, l_end, tq_p, w_p)

    k_p, v_p = _project_kv(h_protein, wk_p, wv_p)
    k_l, v_l = _project_kv(h_ligand, wk_l, wv_l)

    l_final = _attn_update(h_ligand, ligand_batch, protein_batch, k_p, v_p,
                           wq_l, wg_l, bg_l, wu_l, bu_l, ln_l_g, ln_l_b,
                           fl_w1, fl_b1, fl_w2, fl_b2, lohi_l, tq_l, w_l)
    p_final = _attn_update(h_protein, protein_batch, ligand_batch, k_l, v_l,
                           wq_p, wg_p, bg_p, wu_p, bu_p, ln_p_g, ln_p_b,
                           fp_w1, fp_b1, fp_w2, fp_b2, lohi_p, tq_p, w_p)
    return (p_final, l_final)


# speedup vs baseline: 2.9419x; 1.2242x over previous
import math

import jax
import jax.numpy as jnp
from jax.experimental import pallas as pl
from jax.experimental.pallas import tpu as pltpu

FD = 256
HEADS = 8
HDIM = FD // HEADS
NBATCH = 16
_SCALE = 1.0 / math.sqrt(HDIM)
_NEG = -1e30


def _proj_body(h_ref, wk_ref, wv_ref, k_ref, v_ref):
    h = h_ref[...].astype(jnp.bfloat16)
    k_ref[...] = jnp.dot(h, wk_ref[...],
                         preferred_element_type=jnp.float32).astype(jnp.bfloat16)
    v_ref[...] = jnp.dot(h, wv_ref[...],
                         preferred_element_type=jnp.float32).astype(jnp.bfloat16)


def _project_kv(h, wk, wv, tq=256):
    n = h.shape[0]
    bs_h = pl.BlockSpec((tq, FD), lambda i: (i, 0))
    bs_w = pl.BlockSpec((FD, FD), lambda i: (0, 0))
    return pl.pallas_call(
        _proj_body,
        grid=(n // tq,),
        in_specs=[bs_h, bs_w, bs_w],
        out_specs=[bs_h, bs_h],
        out_shape=[jax.ShapeDtypeStruct((n, FD), jnp.bfloat16)] * 2,
    )(h, wk.astype(jnp.bfloat16), wv.astype(jnp.bfloat16))


def _make_attn_body(w):
    def _attn_body(lohi_ref, h_ref, qb_ref, kb_ref, k_ref, v_ref,
                   wq_ref, wgh_ref, wgc_ref, bg_ref, wu_ref, bu_ref,
                   g_ref, b_ref, w1_ref, b1_ref, w2_ref, b2_ref, out_ref,
                   acc_ref):
        i = pl.program_id(0)
        lo = lohi_ref[2 * i]
        hi = lohi_ref[2 * i + 1]
        h = h_ref[...]
        tq = h.shape[0]
        hb = h.astype(jnp.bfloat16)
        q = jnp.dot(hb, wq_ref[...],
                    preferred_element_type=jnp.float32) * _SCALE
        qbf = q.astype(jnp.bfloat16)
        qhs = [qbf[:, hd * HDIM:(hd + 1) * HDIM] for hd in range(HEADS)]
        qb = qb_ref[...]
        acc_ref[...] = jnp.zeros((tq, FD), jnp.float32)

        sls = [slice(hd * HDIM, (hd + 1) * HDIM) for hd in range(HEADS)]

        def body(j, carry):
            ms, ls = carry
            kb = kb_ref[pl.ds(j, 1), :]
            mask = qb == kb
            rows = pl.ds(j * w, w)
            khs = [k_ref[rows, sls[hd]] for hd in range(HEADS)]
            vhs = [v_ref[rows, sls[hd]] for hd in range(HEADS)]
            ss = [jax.lax.dot_general(qhs[hd], khs[hd],
                                      (((1,), (1,)), ((), ())),
                                      preferred_element_type=jnp.float32)
                  for hd in range(HEADS)]
            ss = [jnp.where(mask, s, _NEG) for s in ss]
            nms = [jnp.maximum(ms[hd], ss[hd].max(axis=1, keepdims=True))
                   for hd in range(HEADS)]
            ps = [jnp.exp(ss[hd] - nms[hd]) for hd in range(HEADS)]
            alphas = [jnp.exp(ms[hd] - nms[hd]) for hd in range(HEADS)]
            nls = [ls[hd] * alphas[hd] + ps[hd].sum(axis=1, keepdims=True)
                   for hd in range(HEADS)]
            pvs = [jax.lax.dot_general(ps[hd].astype(jnp.bfloat16), vhs[hd],
                                       (((1,), (0,)), ((), ())),
                                       preferred_element_type=jnp.float32)
                   for hd in range(HEADS)]
            for hd in range(HEADS):
                acc_ref[:, sls[hd]] = acc_ref[:, sls[hd]] * alphas[hd] + pvs[hd]
            return nms, nls

        m0 = [jnp.full((tq, 1), _NEG, jnp.float32)] * HEADS
        l0 = [jnp.zeros((tq, 1), jnp.float32)] * HEADS
        ms, ls = jax.lax.fori_loop(lo, hi, body, (m0, l0))
        ctx = jnp.concatenate(
            [jnp.where(ms[hd] > -1e29,
                       acc_ref[:, sls[hd]] / jnp.where(ls[hd] > 0.0, ls[hd], 1.0),
                       0.0) for hd in range(HEADS)],
            axis=1)
        ctxb = ctx.astype(jnp.bfloat16)

        gate = jax.nn.sigmoid(
            jnp.dot(hb, wgh_ref[...], preferred_element_type=jnp.float32)
            + jnp.dot(ctxb, wgc_ref[...], preferred_element_type=jnp.float32)
            + bg_ref[...])
        hu = h + gate * (jnp.dot(ctxb, wu_ref[...],
                                 preferred_element_type=jnp.float32)
                         + bu_ref[...])
        mean = jnp.mean(hu, axis=1, keepdims=True)
        var = jnp.mean((hu - mean) ** 2, axis=1, keepdims=True)
        y = ((hu - mean) / jnp.sqrt(var + 1e-5) * g_ref[...]
             + b_ref[...]).astype(jnp.bfloat16)
        out = hu
        for c in range(4):
            cs = slice(c * FD, (c + 1) * FD)
            z = jnp.maximum(
                jnp.dot(y, w1_ref[:, cs], preferred_element_type=jnp.float32)
                + b1_ref[:, cs], 0.0)
            out = out + jnp.dot(z.astype(jnp.bfloat16), w2_ref[cs, :],
                                preferred_element_type=jnp.float32)
        out_ref[...] = out + b2_ref[...]

    return _attn_body


def _attn_update(h, q_batch, k_batch, kmat, vmat, wq, wg, bg, wu, bu,
                 ln_g, ln_b, w1, b1, w2, b2, lohi, tq, w):
    nq = h.shape[0]
    nk = kmat.shape[0]
    nkt = nk // w
    bf = jnp.bfloat16
    wgh = wg[:FD].astype(bf)
    wgc = wg[FD:].astype(bf)
    full = lambda shape: pl.BlockSpec(shape, lambda i, s: (0, 0))
    grid_spec = pltpu.PrefetchScalarGridSpec(
        num_scalar_prefetch=1,
        grid=(nq // tq,),
        in_specs=[
            pl.BlockSpec((tq, FD), lambda i, s: (i, 0)),
            pl.BlockSpec((tq, 1), lambda i, s: (i, 0)),
            full((nkt, w)),
            full((nk, FD)),
            full((nk, FD)),
            full((FD, FD)),
            full((FD, FD)),
            full((FD, FD)),
            full((1, FD)),
            full((FD, FD)),
            full((1, FD)),
            full((1, FD)),
            full((1, FD)),
            full((FD, 4 * FD)),
            full((1, 4 * FD)),
            full((4 * FD, FD)),
            full((1, FD)),
        ],
        out_specs=pl.BlockSpec((tq, FD), lambda i, s: (i, 0)),
        scratch_shapes=[pltpu.VMEM((tq, FD), jnp.float32)],
    )
    return pl.pallas_call(
        _make_attn_body(w),
        grid_spec=grid_spec,
        out_shape=jax.ShapeDtypeStruct((nq, FD), jnp.float32),
    )(lohi, h, q_batch.reshape(nq, 1), k_batch.reshape(nkt, w), kmat, vmat,
      wq.astype(bf), wgh, wgc, bg.reshape(1, FD), wu.astype(bf),
      bu.reshape(1, FD), ln_g.reshape(1, FD), ln_b.reshape(1, FD),
      w1.astype(bf), b1.reshape(1, 4 * FD), w2.astype(bf), b2.reshape(1, FD))


def _tile_bounds(q_batch, k_starts, k_ends, tq, w):
    qb = q_batch.reshape(-1, tq)
    bmin = qb[:, 0]
    bmax = qb[:, -1]
    lo = (k_starts[bmin] // w).astype(jnp.int32)
    hi = ((k_ends[bmax] + w - 1) // w).astype(jnp.int32)
    return jnp.stack([lo, hi], axis=1).reshape(-1)


def kernel(h_protein, h_ligand, protein_batch, ligand_batch, wq_l, wk_p, wv_p,
           wg_l, bg_l, wu_l, bu_l, wq_p, wk_l, wv_l, wg_p, bg_p, wu_p, bu_p,
           ln_p_g, ln_p_b, ln_l_g, ln_l_b, fp_w1, fp_b1, fp_w2, fp_b2,
           fl_w1, fl_b1, fl_w2, fl_b2):
    bvec = jnp.arange(NBATCH, dtype=protein_batch.dtype)
    p_start = jnp.sum(protein_batch[None, :] < bvec[:, None], axis=1)
    p_end = jnp.sum(protein_batch[None, :] <= bvec[:, None], axis=1)
    l_start = jnp.sum(ligand_batch[None, :] < bvec[:, None], axis=1)
    l_end = jnp.sum(ligand_batch[None, :] <= bvec[:, None], axis=1)

    tq_l, w_l = 256, 512
    tq_p, w_p = 256, 256
    lohi_l = _tile_bounds(ligand_batch, p_start, p_end, tq_l, w_l)
    lohi_p = _tile_bounds(protein_batch, l_start, l_end, tq_p, w_p)

    k_p, v_p = _project_kv(h_protein, wk_p, wv_p)
    k_l, v_l = _project_kv(h_ligand, wk_l, wv_l)

    l_final = _attn_update(h_ligand, ligand_batch, protein_batch, k_p, v_p,
                           wq_l, wg_l, bg_l, wu_l, bu_l, ln_l_g, ln_l_b,
                           fl_w1, fl_b1, fl_w2, fl_b2, lohi_l, tq_l, w_l)
    p_final = _attn_update(h_protein, protein_batch, ligand_batch, k_l, v_l,
                           wq_p, wg_p, bg_p, wu_p, bu_p, ln_p_g, ln_p_b,
                           fp_w1, fp_b1, fp_w2, fp_b2, lohi_p, tq_p, w_p)
    return (p_final, l_final)


# speedup vs baseline: 3.1442x; 1.0687x over previous
import math

import jax
import jax.numpy as jnp
from jax.experimental import pallas as pl
from jax.experimental.pallas import tpu as pltpu

FD = 256
HEADS = 8
HDIM = FD // HEADS
NBATCH = 16
_SCALE = 1.0 / math.sqrt(HDIM)
_NEG = -1e30


def _proj_body(h_ref, wk_ref, wv_ref, k_ref, v_ref):
    h = h_ref[...].astype(jnp.bfloat16)
    k_ref[...] = jnp.dot(h, wk_ref[...],
                         preferred_element_type=jnp.float32).astype(jnp.bfloat16)
    v_ref[...] = jnp.dot(h, wv_ref[...],
                         preferred_element_type=jnp.float32).astype(jnp.bfloat16)


def _project_kv(h, wk, wv, tq=256):
    n = h.shape[0]
    bs_h = pl.BlockSpec((tq, FD), lambda i: (i, 0))
    bs_w = pl.BlockSpec((FD, FD), lambda i: (0, 0))
    return pl.pallas_call(
        _proj_body,
        grid=(n // tq,),
        in_specs=[bs_h, bs_w, bs_w],
        out_specs=[bs_h, bs_h],
        out_shape=[jax.ShapeDtypeStruct((n, FD), jnp.bfloat16)] * 2,
    )(h, wk.astype(jnp.bfloat16), wv.astype(jnp.bfloat16))


def _make_attn_body(w):
    def _attn_body(lohi_ref, h_ref, qb_ref, kb_ref, k_ref, v_ref,
                   wq_ref, wgh_ref, wgc_ref, bg_ref, wu_ref, bu_ref,
                   g_ref, b_ref, w1_ref, b1_ref, w2_ref, b2_ref, out_ref,
                   acc_ref):
        i = pl.program_id(0)
        lo = lohi_ref[2 * i]
        hi = lohi_ref[2 * i + 1]
        h = h_ref[...]
        tq = h.shape[0]
        hb = h.astype(jnp.bfloat16)
        q = jnp.dot(hb, wq_ref[...],
                    preferred_element_type=jnp.float32) * _SCALE
        qbf = q.astype(jnp.bfloat16)
        qhs = [qbf[:, hd * HDIM:(hd + 1) * HDIM] for hd in range(HEADS)]
        qb = qb_ref[...]
        gh = jnp.dot(hb, wgh_ref[...], preferred_element_type=jnp.float32)
        acc_ref[...] = jnp.zeros((tq, FD), jnp.float32)

        sls = [slice(hd * HDIM, (hd + 1) * HDIM) for hd in range(HEADS)]

        def body(j, carry):
            ms, ls = carry
            kb = kb_ref[pl.ds(j, 1), :]
            mask = qb == kb
            rows = pl.ds(j * w, w)
            khs = [k_ref[rows, sls[hd]] for hd in range(HEADS)]
            vhs = [v_ref[rows, sls[hd]] for hd in range(HEADS)]
            ss = [jax.lax.dot_general(qhs[hd], khs[hd],
                                      (((1,), (1,)), ((), ())),
                                      preferred_element_type=jnp.float32)
                  for hd in range(HEADS)]
            ss = [jnp.where(mask, s, _NEG) for s in ss]
            nms = [jnp.maximum(ms[hd], ss[hd].max(axis=1, keepdims=True))
                   for hd in range(HEADS)]
            ps = [jnp.exp(ss[hd] - nms[hd]) for hd in range(HEADS)]
            alphas = [jnp.exp(ms[hd] - nms[hd]) for hd in range(HEADS)]
            nls = [ls[hd] * alphas[hd] + ps[hd].sum(axis=1, keepdims=True)
                   for hd in range(HEADS)]
            pvs = [jax.lax.dot_general(ps[hd].astype(jnp.bfloat16), vhs[hd],
                                       (((1,), (0,)), ((), ())),
                                       preferred_element_type=jnp.float32)
                   for hd in range(HEADS)]
            for hd in range(HEADS):
                acc_ref[:, sls[hd]] = acc_ref[:, sls[hd]] * alphas[hd] + pvs[hd]
            return nms, nls

        m0 = [jnp.full((tq, 1), _NEG, jnp.float32)] * HEADS
        l0 = [jnp.zeros((tq, 1), jnp.float32)] * HEADS
        ms, ls = jax.lax.fori_loop(lo, hi, body, (m0, l0))
        ctx = jnp.concatenate(
            [jnp.where(ms[hd] > -1e29,
                       acc_ref[:, sls[hd]] / jnp.where(ls[hd] > 0.0, ls[hd], 1.0),
                       0.0) for hd in range(HEADS)],
            axis=1)
        ctxb = ctx.astype(jnp.bfloat16)

        gate = jax.nn.sigmoid(
            gh + jnp.dot(ctxb, wgc_ref[...], preferred_element_type=jnp.float32)
            + bg_ref[...])
        hu = h + gate * (jnp.dot(ctxb, wu_ref[...],
                                 preferred_element_type=jnp.float32)
                         + bu_ref[...])
        mean = jnp.mean(hu, axis=1, keepdims=True)
        var = jnp.mean(hu * hu, axis=1, keepdims=True) - mean * mean
        y = ((hu - mean) / jnp.sqrt(var + 1e-5) * g_ref[...]
             + b_ref[...]).astype(jnp.bfloat16)
        out = hu
        for c in range(4):
            cs = slice(c * FD, (c + 1) * FD)
            z = jnp.maximum(
                jnp.dot(y, w1_ref[:, cs], preferred_element_type=jnp.float32)
                + b1_ref[:, cs], 0.0)
            out = out + jnp.dot(z.astype(jnp.bfloat16), w2_ref[cs, :],
                                preferred_element_type=jnp.float32)
        out_ref[...] = out + b2_ref[...]

    return _attn_body


def _attn_update(h, q_batch, k_batch, kmat, vmat, wq, wg, bg, wu, bu,
                 ln_g, ln_b, w1, b1, w2, b2, lohi, tq, w):
    nq = h.shape[0]
    nk = kmat.shape[0]
    nkt = nk // w
    bf = jnp.bfloat16
    wgh = wg[:FD].astype(bf)
    wgc = wg[FD:].astype(bf)
    full = lambda shape: pl.BlockSpec(shape, lambda i, s: (0, 0))
    grid_spec = pltpu.PrefetchScalarGridSpec(
        num_scalar_prefetch=1,
        grid=(nq // tq,),
        in_specs=[
            pl.BlockSpec((tq, FD), lambda i, s: (i, 0)),
            pl.BlockSpec((tq, 1), lambda i, s: (i, 0)),
            full((nkt, w)),
            full((nk, FD)),
            full((nk, FD)),
            full((FD, FD)),
            full((FD, FD)),
            full((FD, FD)),
            full((1, FD)),
            full((FD, FD)),
            full((1, FD)),
            full((1, FD)),
            full((1, FD)),
            full((FD, 4 * FD)),
            full((1, 4 * FD)),
            full((4 * FD, FD)),
            full((1, FD)),
        ],
        out_specs=pl.BlockSpec((tq, FD), lambda i, s: (i, 0)),
        scratch_shapes=[pltpu.VMEM((tq, FD), jnp.float32)],
    )
    return pl.pallas_call(
        _make_attn_body(w),
        grid_spec=grid_spec,
        out_shape=jax.ShapeDtypeStruct((nq, FD), jnp.float32),
    )(lohi, h, q_batch.reshape(nq, 1), k_batch.reshape(nkt, w), kmat, vmat,
      wq.astype(bf), wgh, wgc, bg.reshape(1, FD), wu.astype(bf),
      bu.reshape(1, FD), ln_g.reshape(1, FD), ln_b.reshape(1, FD),
      w1.astype(bf), b1.reshape(1, 4 * FD), w2.astype(bf), b2.reshape(1, FD))


def _tile_bounds(q_batch, k_starts, k_ends, tq, w):
    qb = q_batch.reshape(-1, tq)
    bmin = qb[:, 0]
    bmax = qb[:, -1]
    lo = (k_starts[bmin] // w).astype(jnp.int32)
    hi = ((k_ends[bmax] + w - 1) // w).astype(jnp.int32)
    return jnp.stack([lo, hi], axis=1).reshape(-1)


def kernel(h_protein, h_ligand, protein_batch, ligand_batch, wq_l, wk_p, wv_p,
           wg_l, bg_l, wu_l, bu_l, wq_p, wk_l, wv_l, wg_p, bg_p, wu_p, bu_p,
           ln_p_g, ln_p_b, ln_l_g, ln_l_b, fp_w1, fp_b1, fp_w2, fp_b2,
           fl_w1, fl_b1, fl_w2, fl_b2):
    bvec = jnp.arange(NBATCH, dtype=protein_batch.dtype)
    p_start = jnp.sum(protein_batch[None, :] < bvec[:, None], axis=1)
    p_end = jnp.sum(protein_batch[None, :] <= bvec[:, None], axis=1)
    l_start = jnp.sum(ligand_batch[None, :] < bvec[:, None], axis=1)
    l_end = jnp.sum(ligand_batch[None, :] <= bvec[:, None], axis=1)

    tq_l, w_l = 256, 512
    tq_p, w_p = 512, 256
    lohi_l = _tile_bounds(ligand_batch, p_start, p_end, tq_l, w_l)
    lohi_p = _tile_bounds(protein_batch, l_start, l_end, tq_p, w_p)

    k_p, v_p = _project_kv(h_protein, wk_p, wv_p)
    k_l, v_l = _project_kv(h_ligand, wk_l, wv_l)

    l_final = _attn_update(h_ligand, ligand_batch, protein_batch, k_p, v_p,
                           wq_l, wg_l, bg_l, wu_l, bu_l, ln_l_g, ln_l_b,
                           fl_w1, fl_b1, fl_w2, fl_b2, lohi_l, tq_l, w_l)
    p_final = _attn_update(h_protein, protein_batch, ligand_batch, k_l, v_l,
                           wq_p, wg_p, bg_p, wu_p, bu_p, ln_p_g, ln_p_b,
                           fp_w1, fp_b1, fp_w2, fp_b2, lohi_p, tq_p, w_p)
    return (p_final, l_final)
